# baseline (speedup 1.0000x reference)
import functools

import jax
import jax.numpy as jnp
from jax import lax
from jax.experimental import pallas as pl
from jax.experimental.pallas import tpu as pltpu
from jax.experimental.pallas import tpu_sc as plsc

N = 10000
D = 128
DH = 64
N_CLS = 40
NP = 10112
E = 320000
EP = 327680
EPT = EP // 16
C = 128
STRIPE = NP // 16


_NCH = EPT // C
_K = 5
_CHS = 40


def _make_sc_agg(with_deg: bool, nch: int = _NCH):
    mesh = plsc.VectorSubcoreMesh(core_axis_name="c", subcore_axis_name="s")
    out_type = [jax.ShapeDtypeStruct((NP, DH), jnp.float32),
                jax.ShapeDtypeStruct((NP, DH), jnp.float32)]
    if with_deg:
        out_type.append(jax.ShapeDtypeStruct((NP, 16), jnp.float32))
    scratch = (
        [pltpu.VMEM((_CHS, C), jnp.int32),
         pltpu.VMEM((_CHS, C), jnp.int32)]
        + [pltpu.VMEM((C, DH), jnp.float32)] * _K
        + [pltpu.VMEM_SHARED((NP, DH), jnp.float32)]
        + ([pltpu.VMEM((C, 16), jnp.float32),
            pltpu.VMEM((STRIPE, 16), jnp.float32),
            pltpu.VMEM_SHARED((NP, 16), jnp.float32)]
           if with_deg else [])
        + [pltpu.SemaphoreType.DMA] * (2 * _K)
    )

    @functools.partial(
        pl.kernel, mesh=mesh, out_type=out_type, scratch_types=scratch,
        compiler_params=pltpu.CompilerParams(use_tc_tiling_on_sc=False))
    def sc_agg(ycat_hbm, src_hbm, dst_hbm, zeros_hbm, zeros16_hbm, ones_hbm,
               *refs):
        if with_deg:
            out0, out1, dout = refs[:3]
            refs = refs[3:]
        else:
            out0, out1 = refs[:2]
            dout = None
            refs = refs[2:]
        idxs, idxd = refs[0], refs[1]
        rows = refs[2:2 + _K]
        acc = refs[2 + _K]
        refs = refs[3 + _K:]
        if with_deg:
            onesv, zbuf16, dacc = refs[:3]
            refs = refs[3:]
        else:
            onesv = zbuf16 = dacc = None
        gsem = refs[:_K]
        ssem = refs[_K:2 * _K]
        cid = lax.axis_index("c")
        sid = lax.axis_index("s")
        wid = cid * 16 + sid
        row0 = sid * STRIPE

        pltpu.sync_copy(zeros_hbm.at[pl.ds(0, C)], rows[0])
        for t in range(4):
            pltpu.sync_copy(rows[0], acc.at[pl.ds(row0 + t * C, C)])
        pltpu.sync_copy(rows[0].at[pl.ds(0, STRIPE - 4 * C)],
                        acc.at[pl.ds(row0 + 4 * C, STRIPE - 4 * C)])
        if with_deg:
            pltpu.sync_copy(zeros16_hbm.at[pl.ds(row0, STRIPE)], zbuf16)
            pltpu.sync_copy(zbuf16, dacc.at[pl.ds(row0, STRIPE)])
            pltpu.sync_copy(ones_hbm, onesv)
        plsc.subcore_barrier()

        for s in range(nch // _CHS):
            pltpu.sync_copy(src_hbm.at[wid, pl.ds(s * _CHS, _CHS)], idxs)
            pltpu.sync_copy(dst_hbm.at[wid, pl.ds(s * _CHS, _CHS)], idxd)

            @pl.loop(0, _CHS // _K)
            def edge_group(g):
                ch0 = g * _K
                gd = [pltpu.async_copy(ycat_hbm.at[idxs.at[ch0 + j]],
                                       rows[j], gsem[j]) for j in range(_K)]
                sd = []
                for j in range(_K):
                    gd[j].wait()
                    sd.append(pltpu.async_copy(rows[j],
                                               acc.at[idxd.at[ch0 + j]],
                                               ssem[j], add=True))
                    if with_deg:
                        @pl.when(cid == 0)
                        def _(j=j):
                            pltpu.sync_copy(onesv, dacc.at[idxd.at[ch0 + j]],
                                            add=True)
                for j in range(_K):
                    sd[j].wait()

        plsc.subcore_barrier()

        def writeout(out):
            tail = STRIPE - 4 * C
            rd = [pltpu.async_copy(acc.at[pl.ds(row0 + t * C, C)],
                                   rows[t], gsem[t]) for t in range(4)]
            for t in range(4):
                rd[t].wait()
                pltpu.sync_copy(rows[t], out.at[pl.ds(row0 + t * C, C)])
            pltpu.sync_copy(acc.at[pl.ds(row0 + 4 * C, tail)],
                            rows[0].at[pl.ds(0, tail)])
            pltpu.sync_copy(rows[0].at[pl.ds(0, tail)],
                            out.at[pl.ds(row0 + 4 * C, tail)])

        @pl.when(cid == 0)
        def _():
            writeout(out0)
            if with_deg:
                pltpu.sync_copy(dacc.at[pl.ds(row0, STRIPE)], zbuf16)
                pltpu.sync_copy(zbuf16, dout.at[pl.ds(row0, STRIPE)])

        @pl.when(cid == 1)
        def _():
            writeout(out1)

    return sc_agg


_sc_agg_deg = _make_sc_agg(True)
_sc_agg = _make_sc_agg(False)
_sc_agg_es = _make_sc_agg(False, nch=EP // 32 // C)


_BLK = 1000


def _mm_body(x_ref, w_ref, o_ref):
    o_ref[...] = jnp.dot(x_ref[...], w_ref[...],
                         preferred_element_type=jnp.float32)


def _tc_project(x, w):
    n, k = x.shape
    m = w.shape[1]
    return pl.pallas_call(
        _mm_body,
        grid=(n // _BLK,),
        in_specs=[pl.BlockSpec((_BLK, k), lambda i: (i, 0)),
                  pl.BlockSpec((k, m), lambda i: (0, 0))],
        out_specs=pl.BlockSpec((_BLK, m), lambda i: (i, 0)),
        out_shape=jax.ShapeDtypeStruct((NP, m), jnp.float32),
    )(x, w)


def _combine_body(pl_ref, pr_ref, d_ref, h_ref, wr_ref, b_ref,
                  wl_ref, h_out, y_out):
    deg = d_ref[:, 0:1]
    rinv = 1.0 / jnp.maximum(deg, 1.0)
    mean_wl = jnp.concatenate([pl_ref[...], pr_ref[...]], axis=1) * rinv
    hn = mean_wl + jnp.dot(h_ref[...], wr_ref[...],
                           preferred_element_type=jnp.float32) + b_ref[0:1, :]
    hn = jnp.maximum(hn, 0.0)
    h_out[...] = hn
    y_out[...] = jnp.dot(hn, wl_ref[...], preferred_element_type=jnp.float32)


def _tc_combine(p0, p1, d, h, wr, b8, wl_next):
    m = wl_next.shape[1]
    return pl.pallas_call(
        _combine_body,
        grid=(N // _BLK,),
        in_specs=[
            pl.BlockSpec((_BLK, DH), lambda i: (i, 0)),
            pl.BlockSpec((_BLK, DH), lambda i: (i, 0)),
            pl.BlockSpec((_BLK, 16), lambda i: (i, 0)),
            pl.BlockSpec((_BLK, D), lambda i: (i, 0)),
            pl.BlockSpec((D, D), lambda i: (0, 0)),
            pl.BlockSpec((8, D), lambda i: (0, 0)),
            pl.BlockSpec((D, m), lambda i: (0, 0)),
        ],
        out_specs=[pl.BlockSpec((_BLK, D), lambda i: (i, 0)),
                   pl.BlockSpec((_BLK, m), lambda i: (i, 0))],
        out_shape=[jax.ShapeDtypeStruct((N, D), jnp.float32),
                   jax.ShapeDtypeStruct((NP, m), jnp.float32)],
    )(p0, p1, d, h, wr, b8, wl_next)


def _final_body(p0_ref, p1_ref, d_ref, h_ref, wr_ref, b_ref, o_ref):
    deg = d_ref[:, 0:1]
    rinv = 1.0 / jnp.maximum(deg, 1.0)
    mean_wl = (p0_ref[...] + p1_ref[...]) * rinv
    o_ref[...] = (mean_wl[:, :N_CLS]
                  + jnp.dot(h_ref[...], wr_ref[...],
                            preferred_element_type=jnp.float32)
                  + b_ref[0:1, :])


def _tc_final(p0, p1, d, h, wr, b8):
    return pl.pallas_call(
        _final_body,
        grid=(N // _BLK,),
        in_specs=[
            pl.BlockSpec((_BLK, DH), lambda i: (i, 0)),
            pl.BlockSpec((_BLK, DH), lambda i: (i, 0)),
            pl.BlockSpec((_BLK, 16), lambda i: (i, 0)),
            pl.BlockSpec((_BLK, D), lambda i: (i, 0)),
            pl.BlockSpec((D, N_CLS), lambda i: (0, 0)),
            pl.BlockSpec((8, N_CLS), lambda i: (0, 0)),
        ],
        out_specs=pl.BlockSpec((_BLK, N_CLS), lambda i: (i, 0)),
        out_shape=jax.ShapeDtypeStruct((N, N_CLS), jnp.float32),
    )(p0, p1, d, h, wr, b8)


def kernel(x, edge_index, Wl0, Wr0, b0, Wl1, Wr1, b1, Wl2, Wr2, b2):
    pad_e = EP - E
    src = jnp.concatenate(
        [edge_index[0], jnp.full((pad_e,), N, dtype=jnp.int32)])
    dst = jnp.concatenate(
        [edge_index[1], jnp.full((pad_e,), N, dtype=jnp.int32)])
    src_big = jnp.concatenate([src * 2, src * 2 + 1]).reshape(32, _NCH, C)
    dst_big = jnp.concatenate([dst, dst]).reshape(32, _NCH, C)
    src_es = src.reshape(32, _NCH // 2, C)
    dst_es = dst.reshape(32, _NCH // 2, C)
    zeros = jnp.zeros((NP, DH), jnp.float32)
    zeros16 = jnp.zeros((NP, 16), jnp.float32)
    ones = jnp.ones((C, 16), jnp.float32)
    b0_8 = jnp.broadcast_to(b0, (8, D))
    b1_8 = jnp.broadcast_to(b1, (8, D))
    b2_8 = jnp.broadcast_to(b2, (8, N_CLS))
    Wl2p = jnp.pad(Wl2, ((0, 0), (0, DH - N_CLS)))

    y0 = _tc_project(x, Wl0)
    pL, pR, dp = _sc_agg_deg(y0.reshape(2 * NP, DH), src_big, dst_big,
                             zeros, zeros16, ones)
    h1, y1 = _tc_combine(pL, pR, dp, x, Wr0, b0_8, Wl1)

    pL, pR = _sc_agg(y1.reshape(2 * NP, DH), src_big, dst_big, zeros,
                     zeros16, ones)
    h2, y2 = _tc_combine(pL, pR, dp, h1, Wr1, b1_8, Wl2p)

    pA, pB = _sc_agg_es(y2, src_es, dst_es, zeros, zeros16, ones)
    out = _tc_final(pA, pB, dp, h2, Wr2, b2_8)
    return out


# speedup vs baseline: 1.1674x; 1.1674x over previous
import functools

import jax
import jax.numpy as jnp
from jax import lax
from jax.experimental import pallas as pl
from jax.experimental.pallas import tpu as pltpu
from jax.experimental.pallas import tpu_sc as plsc

N = 10000
D = 128
DH = 64
N_CLS = 40
NP = 10112
E = 320000
EP = 327680
EPT = EP // 16
C = 128
STRIPE = NP // 16


_NCH = EPT // C
_K = 5
_CHS = 40


def _make_sc_agg(with_deg: bool, nch: int = _NCH):
    mesh = plsc.VectorSubcoreMesh(core_axis_name="c", subcore_axis_name="s")
    out_type = [jax.ShapeDtypeStruct((NP, DH), jnp.float32),
                jax.ShapeDtypeStruct((NP, DH), jnp.float32)]
    if with_deg:
        out_type.append(jax.ShapeDtypeStruct((NP, 16), jnp.float32))
    scratch = (
        [pltpu.VMEM((_CHS, C), jnp.int32),
         pltpu.VMEM((_CHS, C), jnp.int32)]
        + [pltpu.VMEM((C, DH), jnp.float32)] * _K
        + [pltpu.VMEM_SHARED((NP, DH), jnp.float32)]
        + ([pltpu.VMEM((C, 16), jnp.float32),
            pltpu.VMEM((STRIPE, 16), jnp.float32),
            pltpu.VMEM_SHARED((NP, 16), jnp.float32)]
           if with_deg else [])
        + [pltpu.SemaphoreType.DMA] * (2 * _K)
    )

    @functools.partial(
        pl.kernel, mesh=mesh, out_type=out_type, scratch_types=scratch,
        compiler_params=pltpu.CompilerParams(use_tc_tiling_on_sc=False))
    def sc_agg(ycat_hbm, src_hbm, dst_hbm, zeros_hbm, zeros16_hbm, ones_hbm,
               *refs):
        if with_deg:
            out0, out1, dout = refs[:3]
            refs = refs[3:]
        else:
            out0, out1 = refs[:2]
            dout = None
            refs = refs[2:]
        idxs, idxd = refs[0], refs[1]
        rows = refs[2:2 + _K]
        acc = refs[2 + _K]
        refs = refs[3 + _K:]
        if with_deg:
            onesv, zbuf16, dacc = refs[:3]
            refs = refs[3:]
        else:
            onesv = zbuf16 = dacc = None
        gsem = refs[:_K]
        ssem = refs[_K:2 * _K]
        cid = lax.axis_index("c")
        sid = lax.axis_index("s")
        wid = cid * 16 + sid
        row0 = sid * STRIPE

        pltpu.sync_copy(zeros_hbm.at[pl.ds(0, C)], rows[0])
        for t in range(4):
            pltpu.sync_copy(rows[0], acc.at[pl.ds(row0 + t * C, C)])
        pltpu.sync_copy(rows[0].at[pl.ds(0, STRIPE - 4 * C)],
                        acc.at[pl.ds(row0 + 4 * C, STRIPE - 4 * C)])
        if with_deg:
            pltpu.sync_copy(zeros16_hbm.at[pl.ds(row0, STRIPE)], zbuf16)
            pltpu.sync_copy(zbuf16, dacc.at[pl.ds(row0, STRIPE)])
            pltpu.sync_copy(ones_hbm, onesv)
        plsc.subcore_barrier()

        for s in range(nch // _CHS):
            pltpu.sync_copy(src_hbm.at[wid, pl.ds(s * _CHS, _CHS)], idxs)
            pltpu.sync_copy(dst_hbm.at[wid, pl.ds(s * _CHS, _CHS)], idxd)

            @pl.loop(0, _CHS // _K)
            def edge_group(g):
                ch0 = g * _K
                gd = [pltpu.async_copy(ycat_hbm.at[idxs.at[ch0 + j]],
                                       rows[j], gsem[j]) for j in range(_K)]
                sd = []
                for j in range(_K):
                    gd[j].wait()
                    sd.append(pltpu.async_copy(rows[j],
                                               acc.at[idxd.at[ch0 + j]],
                                               ssem[j], add=True))
                    if with_deg:
                        @pl.when(cid == 0)
                        def _(j=j):
                            pltpu.sync_copy(onesv, dacc.at[idxd.at[ch0 + j]],
                                            add=True)
                for j in range(_K):
                    sd[j].wait()

        plsc.subcore_barrier()

        def writeout(out):
            tail = STRIPE - 4 * C
            rd = [pltpu.async_copy(acc.at[pl.ds(row0 + t * C, C)],
                                   rows[t], gsem[t]) for t in range(4)]
            for t in range(4):
                rd[t].wait()
                pltpu.sync_copy(rows[t], out.at[pl.ds(row0 + t * C, C)])
            pltpu.sync_copy(acc.at[pl.ds(row0 + 4 * C, tail)],
                            rows[0].at[pl.ds(0, tail)])
            pltpu.sync_copy(rows[0].at[pl.ds(0, tail)],
                            out.at[pl.ds(row0 + 4 * C, tail)])

        @pl.when(cid == 0)
        def _():
            writeout(out0)
            if with_deg:
                pltpu.sync_copy(dacc.at[pl.ds(row0, STRIPE)], zbuf16)
                pltpu.sync_copy(zbuf16, dout.at[pl.ds(row0, STRIPE)])

        @pl.when(cid == 1)
        def _():
            writeout(out1)

    return sc_agg


_sc_agg_deg = _make_sc_agg(True)
_sc_agg = _make_sc_agg(False)
_sc_agg_es = _make_sc_agg(False, nch=EP // 32 // C)


_BLK = 1000


def _mm_body(x_ref, w_ref, o_ref):
    o_ref[...] = jnp.dot(x_ref[...], w_ref[...],
                         preferred_element_type=jnp.float32)


def _tc_project(x, w):
    n, k = x.shape
    m = w.shape[1]
    return pl.pallas_call(
        _mm_body,
        grid=(n // _BLK,),
        in_specs=[pl.BlockSpec((_BLK, k), lambda i: (i, 0)),
                  pl.BlockSpec((k, m), lambda i: (0, 0))],
        out_specs=pl.BlockSpec((_BLK, m), lambda i: (i, 0)),
        out_shape=jax.ShapeDtypeStruct((n, m), jnp.float32),
    )(x, w)


def _combine_body(pl_ref, pr_ref, d_ref, h_ref, wr_ref, b_ref,
                  wl_ref, h_out, y_out):
    deg = d_ref[:, 0:1]
    rinv = 1.0 / jnp.maximum(deg, 1.0)
    mean_wl = jnp.concatenate([pl_ref[...], pr_ref[...]], axis=1) * rinv
    hn = mean_wl + jnp.dot(h_ref[...], wr_ref[...],
                           preferred_element_type=jnp.float32) + b_ref[0:1, :]
    hn = jnp.maximum(hn, 0.0)
    h_out[...] = hn
    y_out[...] = jnp.dot(hn, wl_ref[...], preferred_element_type=jnp.float32)


def _tc_combine(p0, p1, d, h, wr, b8, wl_next):
    m = wl_next.shape[1]
    return pl.pallas_call(
        _combine_body,
        grid=(N // _BLK,),
        in_specs=[
            pl.BlockSpec((_BLK, DH), lambda i: (i, 0)),
            pl.BlockSpec((_BLK, DH), lambda i: (i, 0)),
            pl.BlockSpec((_BLK, 16), lambda i: (i, 0)),
            pl.BlockSpec((_BLK, D), lambda i: (i, 0)),
            pl.BlockSpec((D, D), lambda i: (0, 0)),
            pl.BlockSpec((8, D), lambda i: (0, 0)),
            pl.BlockSpec((D, m), lambda i: (0, 0)),
        ],
        out_specs=[pl.BlockSpec((_BLK, D), lambda i: (i, 0)),
                   pl.BlockSpec((_BLK, m), lambda i: (i, 0))],
        out_shape=[jax.ShapeDtypeStruct((N, D), jnp.float32),
                   jax.ShapeDtypeStruct((N, m), jnp.float32)],
    )(p0, p1, d, h, wr, b8, wl_next)


def _final_body(p0_ref, p1_ref, d_ref, h_ref, wr_ref, b_ref, o_ref):
    deg = d_ref[:, 0:1]
    rinv = 1.0 / jnp.maximum(deg, 1.0)
    mean_wl = (p0_ref[...] + p1_ref[...]) * rinv
    o_ref[...] = (mean_wl[:, :N_CLS]
                  + jnp.dot(h_ref[...], wr_ref[...],
                            preferred_element_type=jnp.float32)
                  + b_ref[0:1, :])


def _tc_final(p0, p1, d, h, wr, b8):
    return pl.pallas_call(
        _final_body,
        grid=(N // _BLK,),
        in_specs=[
            pl.BlockSpec((_BLK, DH), lambda i: (i, 0)),
            pl.BlockSpec((_BLK, DH), lambda i: (i, 0)),
            pl.BlockSpec((_BLK, 16), lambda i: (i, 0)),
            pl.BlockSpec((_BLK, D), lambda i: (i, 0)),
            pl.BlockSpec((D, N_CLS), lambda i: (0, 0)),
            pl.BlockSpec((8, N_CLS), lambda i: (0, 0)),
        ],
        out_specs=pl.BlockSpec((_BLK, N_CLS), lambda i: (i, 0)),
        out_shape=jax.ShapeDtypeStruct((N, N_CLS), jnp.float32),
    )(p0, p1, d, h, wr, b8)


def _split_pad(y):
    pad = ((0, NP - N), (0, 0))
    return jnp.concatenate([jnp.pad(y[:, :DH], pad),
                            jnp.pad(y[:, DH:], pad)], axis=0)


def kernel(x, edge_index, Wl0, Wr0, b0, Wl1, Wr1, b1, Wl2, Wr2, b2):
    pad_e = EP - E
    src = jnp.concatenate(
        [edge_index[0], jnp.full((pad_e,), N, dtype=jnp.int32)])
    dst = jnp.concatenate(
        [edge_index[1], jnp.full((pad_e,), N, dtype=jnp.int32)])
    src_big = jnp.concatenate([src, src + NP]).reshape(32, _NCH, C)
    dst_big = jnp.concatenate([dst, dst]).reshape(32, _NCH, C)
    src_es = src.reshape(32, _NCH // 2, C)
    dst_es = dst.reshape(32, _NCH // 2, C)
    zeros = jnp.zeros((NP, DH), jnp.float32)
    zeros16 = jnp.zeros((NP, 16), jnp.float32)
    ones = jnp.ones((C, 16), jnp.float32)
    b0_8 = jnp.broadcast_to(b0, (8, D))
    b1_8 = jnp.broadcast_to(b1, (8, D))
    b2_8 = jnp.broadcast_to(b2, (8, N_CLS))
    Wl2p = jnp.pad(Wl2, ((0, 0), (0, DH - N_CLS)))

    y0 = _tc_project(x, Wl0)
    pL, pR, dp = _sc_agg_deg(_split_pad(y0), src_big, dst_big, zeros,
                             zeros16, ones)
    h1, y1 = _tc_combine(pL, pR, dp, x, Wr0, b0_8, Wl1)

    pL, pR = _sc_agg(_split_pad(y1), src_big, dst_big, zeros, zeros16, ones)
    h2, y2 = _tc_combine(pL, pR, dp, h1, Wr1, b1_8, Wl2p)

    y2p = jnp.pad(y2, ((0, NP - N), (0, 0)))
    pA, pB = _sc_agg_es(y2p, src_es, dst_es, zeros, zeros16, ones)
    out = _tc_final(pA, pB, dp, h2, Wr2, b2_8)
    return out


# speedup vs baseline: 1.1683x; 1.0008x over previous
import functools

import jax
import jax.numpy as jnp
from jax import lax
from jax.experimental import pallas as pl
from jax.experimental.pallas import tpu as pltpu
from jax.experimental.pallas import tpu_sc as plsc

N = 10000
D = 128
DH = 64
N_CLS = 40
NP = 10112
E = 320000
EP = 327680
EPT = EP // 16
C = 128
STRIPE = NP // 16


_NCH = EPT // C
_K = 5
_CHS = 40


def _make_sc_agg(with_deg: bool, nch: int = _NCH):
    mesh = plsc.VectorSubcoreMesh(core_axis_name="c", subcore_axis_name="s")
    out_type = [jax.ShapeDtypeStruct((NP, DH), jnp.float32),
                jax.ShapeDtypeStruct((NP, DH), jnp.float32)]
    if with_deg:
        out_type.append(jax.ShapeDtypeStruct((NP, 16), jnp.float32))
    scratch = (
        [pltpu.VMEM((_CHS, C), jnp.int32),
         pltpu.VMEM((_CHS, C), jnp.int32)]
        + [pltpu.VMEM((C, DH), jnp.float32)] * _K
        + [pltpu.VMEM_SHARED((NP, DH), jnp.float32)]
        + ([pltpu.VMEM((C, 16), jnp.float32),
            pltpu.VMEM((STRIPE, 16), jnp.float32),
            pltpu.VMEM_SHARED((NP, 16), jnp.float32)]
           if with_deg else [])
        + [pltpu.SemaphoreType.DMA] * (2 * _K)
    )

    @functools.partial(
        pl.kernel, mesh=mesh, out_type=out_type, scratch_types=scratch,
        compiler_params=pltpu.CompilerParams(use_tc_tiling_on_sc=False))
    def sc_agg(ycat_hbm, src_hbm, dst_hbm, zeros_hbm, zeros16_hbm, ones_hbm,
               *refs):
        if with_deg:
            out0, out1, dout = refs[:3]
            refs = refs[3:]
        else:
            out0, out1 = refs[:2]
            dout = None
            refs = refs[2:]
        idxs, idxd = refs[0], refs[1]
        rows = refs[2:2 + _K]
        acc = refs[2 + _K]
        refs = refs[3 + _K:]
        if with_deg:
            onesv, zbuf16, dacc = refs[:3]
            refs = refs[3:]
        else:
            onesv = zbuf16 = dacc = None
        gsem = refs[:_K]
        ssem = refs[_K:2 * _K]
        cid = lax.axis_index("c")
        sid = lax.axis_index("s")
        wid = cid * 16 + sid
        row0 = sid * STRIPE

        pltpu.sync_copy(zeros_hbm.at[pl.ds(0, C)], rows[0])
        for t in range(4):
            pltpu.sync_copy(rows[0], acc.at[pl.ds(row0 + t * C, C)])
        pltpu.sync_copy(rows[0].at[pl.ds(0, STRIPE - 4 * C)],
                        acc.at[pl.ds(row0 + 4 * C, STRIPE - 4 * C)])
        if with_deg:
            pltpu.sync_copy(zeros16_hbm.at[pl.ds(row0, STRIPE)], zbuf16)
            pltpu.sync_copy(zbuf16, dacc.at[pl.ds(row0, STRIPE)])
            pltpu.sync_copy(ones_hbm, onesv)
        plsc.subcore_barrier()

        for s in range(nch // _CHS):
            pltpu.sync_copy(src_hbm.at[wid, pl.ds(s * _CHS, _CHS)], idxs)
            pltpu.sync_copy(dst_hbm.at[wid, pl.ds(s * _CHS, _CHS)], idxd)

            @pl.loop(0, _CHS // _K)
            def edge_group(g):
                ch0 = g * _K

                @pl.when(g > 0)
                def _():
                    for j in range(_K):
                        pltpu.make_async_copy(
                            rows[j], acc.at[idxd.at[ch0 - _K + j]],
                            ssem[j]).wait()

                gd = [pltpu.async_copy(ycat_hbm.at[idxs.at[ch0 + j]],
                                       rows[j], gsem[j]) for j in range(_K)]
                for j in range(_K):
                    gd[j].wait()
                    pltpu.async_copy(rows[j], acc.at[idxd.at[ch0 + j]],
                                     ssem[j], add=True)
                    if with_deg:
                        @pl.when(cid == 0)
                        def _(j=j):
                            pltpu.sync_copy(onesv, dacc.at[idxd.at[ch0 + j]],
                                            add=True)

            for j in range(_K):
                pltpu.make_async_copy(
                    rows[j], acc.at[idxd.at[_CHS - _K + j]], ssem[j]).wait()

        plsc.subcore_barrier()

        def writeout(out):
            tail = STRIPE - 4 * C
            rd = [pltpu.async_copy(acc.at[pl.ds(row0 + t * C, C)],
                                   rows[t], gsem[t]) for t in range(4)]
            for t in range(4):
                rd[t].wait()
                pltpu.sync_copy(rows[t], out.at[pl.ds(row0 + t * C, C)])
            pltpu.sync_copy(acc.at[pl.ds(row0 + 4 * C, tail)],
                            rows[0].at[pl.ds(0, tail)])
            pltpu.sync_copy(rows[0].at[pl.ds(0, tail)],
                            out.at[pl.ds(row0 + 4 * C, tail)])

        @pl.when(cid == 0)
        def _():
            writeout(out0)
            if with_deg:
                pltpu.sync_copy(dacc.at[pl.ds(row0, STRIPE)], zbuf16)
                pltpu.sync_copy(zbuf16, dout.at[pl.ds(row0, STRIPE)])

        @pl.when(cid == 1)
        def _():
            writeout(out1)

    return sc_agg


_sc_agg_deg = _make_sc_agg(True)
_sc_agg = _make_sc_agg(False)
_sc_agg_es = _make_sc_agg(False, nch=EP // 32 // C)


_BLK = 1000


def _mm_body(x_ref, w_ref, o_ref):
    o_ref[...] = jnp.dot(x_ref[...], w_ref[...],
                         preferred_element_type=jnp.float32)


def _tc_project(x, w):
    n, k = x.shape
    m = w.shape[1]
    return pl.pallas_call(
        _mm_body,
        grid=(n // _BLK,),
        in_specs=[pl.BlockSpec((_BLK, k), lambda i: (i, 0)),
                  pl.BlockSpec((k, m), lambda i: (0, 0))],
        out_specs=pl.BlockSpec((_BLK, m), lambda i: (i, 0)),
        out_shape=jax.ShapeDtypeStruct((n, m), jnp.float32),
    )(x, w)


def _combine_body(pl_ref, pr_ref, d_ref, h_ref, wr_ref, b_ref,
                  wl_ref, h_out, y_out):
    deg = d_ref[:, 0:1]
    rinv = 1.0 / jnp.maximum(deg, 1.0)
    mean_wl = jnp.concatenate([pl_ref[...], pr_ref[...]], axis=1) * rinv
    hn = mean_wl + jnp.dot(h_ref[...], wr_ref[...],
                           preferred_element_type=jnp.float32) + b_ref[0:1, :]
    hn = jnp.maximum(hn, 0.0)
    h_out[...] = hn
    y_out[...] = jnp.dot(hn, wl_ref[...], preferred_element_type=jnp.float32)


def _tc_combine(p0, p1, d, h, wr, b8, wl_next):
    m = wl_next.shape[1]
    return pl.pallas_call(
        _combine_body,
        grid=(N // _BLK,),
        in_specs=[
            pl.BlockSpec((_BLK, DH), lambda i: (i, 0)),
            pl.BlockSpec((_BLK, DH), lambda i: (i, 0)),
            pl.BlockSpec((_BLK, 16), lambda i: (i, 0)),
            pl.BlockSpec((_BLK, D), lambda i: (i, 0)),
            pl.BlockSpec((D, D), lambda i: (0, 0)),
            pl.BlockSpec((8, D), lambda i: (0, 0)),
            pl.BlockSpec((D, m), lambda i: (0, 0)),
        ],
        out_specs=[pl.BlockSpec((_BLK, D), lambda i: (i, 0)),
                   pl.BlockSpec((_BLK, m), lambda i: (i, 0))],
        out_shape=[jax.ShapeDtypeStruct((N, D), jnp.float32),
                   jax.ShapeDtypeStruct((N, m), jnp.float32)],
    )(p0, p1, d, h, wr, b8, wl_next)


def _final_body(p0_ref, p1_ref, d_ref, h_ref, wr_ref, b_ref, o_ref):
    deg = d_ref[:, 0:1]
    rinv = 1.0 / jnp.maximum(deg, 1.0)
    mean_wl = (p0_ref[...] + p1_ref[...]) * rinv
    o_ref[...] = (mean_wl[:, :N_CLS]
                  + jnp.dot(h_ref[...], wr_ref[...],
                            preferred_element_type=jnp.float32)
                  + b_ref[0:1, :])


def _tc_final(p0, p1, d, h, wr, b8):
    return pl.pallas_call(
        _final_body,
        grid=(N // _BLK,),
        in_specs=[
            pl.BlockSpec((_BLK, DH), lambda i: (i, 0)),
            pl.BlockSpec((_BLK, DH), lambda i: (i, 0)),
            pl.BlockSpec((_BLK, 16), lambda i: (i, 0)),
            pl.BlockSpec((_BLK, D), lambda i: (i, 0)),
            pl.BlockSpec((D, N_CLS), lambda i: (0, 0)),
            pl.BlockSpec((8, N_CLS), lambda i: (0, 0)),
        ],
        out_specs=pl.BlockSpec((_BLK, N_CLS), lambda i: (i, 0)),
        out_shape=jax.ShapeDtypeStruct((N, N_CLS), jnp.float32),
    )(p0, p1, d, h, wr, b8)


def _split_pad(y):
    pad = ((0, NP - N), (0, 0))
    return jnp.concatenate([jnp.pad(y[:, :DH], pad),
                            jnp.pad(y[:, DH:], pad)], axis=0)


def kernel(x, edge_index, Wl0, Wr0, b0, Wl1, Wr1, b1, Wl2, Wr2, b2):
    pad_e = EP - E
    src = jnp.concatenate(
        [edge_index[0], jnp.full((pad_e,), N, dtype=jnp.int32)])
    dst = jnp.concatenate(
        [edge_index[1], jnp.full((pad_e,), N, dtype=jnp.int32)])
    src_big = jnp.concatenate([src, src + NP]).reshape(32, _NCH, C)
    dst_big = jnp.concatenate([dst, dst]).reshape(32, _NCH, C)
    src_es = src.reshape(32, _NCH // 2, C)
    dst_es = dst.reshape(32, _NCH // 2, C)
    zeros = jnp.zeros((NP, DH), jnp.float32)
    zeros16 = jnp.zeros((NP, 16), jnp.float32)
    ones = jnp.ones((C, 16), jnp.float32)
    b0_8 = jnp.broadcast_to(b0, (8, D))
    b1_8 = jnp.broadcast_to(b1, (8, D))
    b2_8 = jnp.broadcast_to(b2, (8, N_CLS))
    Wl2p = jnp.pad(Wl2, ((0, 0), (0, DH - N_CLS)))

    y0 = _tc_project(x, Wl0)
    pL, pR, dp = _sc_agg_deg(_split_pad(y0), src_big, dst_big, zeros,
                             zeros16, ones)
    h1, y1 = _tc_combine(pL, pR, dp, x, Wr0, b0_8, Wl1)

    pL, pR = _sc_agg(_split_pad(y1), src_big, dst_big, zeros, zeros16, ones)
    h2, y2 = _tc_combine(pL, pR, dp, h1, Wr1, b1_8, Wl2p)

    y2p = jnp.pad(y2, ((0, NP - N), (0, 0)))
    pA, pB = _sc_agg_es(y2p, src_es, dst_es, zeros, zeros16, ones)
    out = _tc_final(pA, pB, dp, h2, Wr2, b2_8)
    return out
